# baseline (speedup 1.0000x reference)
import functools

import jax
import jax.numpy as jnp
from jax.experimental import pallas as pl
from jax.experimental.pallas import tpu as pltpu

_EPS = 1e-5


def _eye_bf16(n):
    r = jax.lax.broadcasted_iota(jnp.int32, (n, n), 0)
    c = jax.lax.broadcasted_iota(jnp.int32, (n, n), 1)
    return (r == c).astype(jnp.bfloat16)


def _mxu_t(a):
    eye = _eye_bf16(a.shape[0])
    return jax.lax.dot_general(
        a, eye, (((0,), (0,)), ((), ())),
        preferred_element_type=jnp.float32).astype(jnp.bfloat16)


def _fused_spade_kernel(seg_ref, w1_ref, b1_ref, wg_ref, wb_ref, bg_ref,
                        bb_ref, x_ref, o_ref, w1c_ref, w2_ref, mrow_ref,
                        mcol_ref, expand_ref, *, width, halo):
    nb, label_nc, hs, ws = seg_ref.shape
    _, hw, c = x_ref.shape
    nh = w1_ref.shape[1]
    npad = hw + 2 * halo
    fh, fw = (hw // width) // hs, width // ws

    @pl.when(pl.program_id(0) == 0)
    def _prep():
        liota = jax.lax.broadcasted_iota(jnp.int32, (ws, width), 1)
        qiota = jax.lax.broadcasted_iota(jnp.int32, (ws, width), 0)
        expand_ref[...] = (liota // fw == qiota).astype(jnp.bfloat16)
        p = jax.lax.broadcasted_iota(jnp.int32, (1, npad), 1) - halo
        pw = jax.lax.rem(p, width)
        mrow_ref[...] = jnp.concatenate(
            [((p >= 0) & (p < hw)).astype(jnp.bfloat16),
             (pw != 0).astype(jnp.bfloat16),
             jnp.logical_and(pw != width - 1,
                             pw != -1).astype(jnp.bfloat16)], axis=0)
        j = jax.lax.broadcasted_iota(jnp.int32, (hw, 2), 0)
        jw = jax.lax.rem(j, width)
        mcol_ref[...] = jnp.concatenate(
            [(jw[:, 0:1] != 0).astype(jnp.bfloat16),
             (jw[:, 1:2] != width - 1).astype(jnp.bfloat16)], axis=1)
        w1c_ref[...] = jnp.concatenate(
            [w1_ref[k].astype(jnp.bfloat16) for k in range(9)], axis=1)
        wgbc = jnp.concatenate(
            [jnp.concatenate([wg_ref[k].astype(jnp.bfloat16),
                              wb_ref[k].astype(jnp.bfloat16)], axis=0)
             for k in range(9)], axis=1)
        w2_ref[...] = _mxu_t(wgbc)

    mrow = mrow_ref[...]
    interior, nl1, nr1 = mrow[0:1], mrow[1:2], mrow[2:3]
    nl2, nr2 = mcol_ref[:, 0:1], mcol_ref[:, 1:2]
    expand = expand_ref[...]
    w1c = w1c_ref[...]
    w2 = w2_ref[...]

    for bi in range(nb):
        s = seg_ref[bi].astype(jnp.bfloat16)
        d = jnp.dot(s.reshape(label_nc * hs, ws), expand,
                    preferred_element_type=jnp.float32).astype(jnp.bfloat16)
        d = d.reshape(label_nc, hs, width)
        e = jnp.concatenate([d] * fh, axis=2)
        seg = e.reshape(label_nc, hw)
        seg = jnp.pad(seg, ((0, 0), (2 * halo, 2 * halo)))

        taps1 = []
        for kh in range(3):
            for kw in range(3):
                off = (kh - 1) * width + (kw - 1)
                tap = seg[:, halo + off: halo + off + npad]
                if kw == 0:
                    tap = tap * nl1
                elif kw == 2:
                    tap = tap * nr1
                taps1.append(tap)
        col1 = jnp.concatenate(taps1, axis=0)
        acc1 = jnp.dot(w1c, col1, preferred_element_type=jnp.float32)
        actv = jnp.maximum(acc1 + b1_ref[...], 0.0).astype(jnp.bfloat16)
        actv = actv * interior
        actv_t = _mxu_t(actv)

        taps2 = []
        for kh in range(3):
            for kw in range(3):
                off = (kh - 1) * width + (kw - 1)
                tap = actv_t[halo + off: halo + off + hw, :]
                if kw == 0:
                    tap = tap * nl2
                elif kw == 2:
                    tap = tap * nr2
                taps2.append(tap)
        col2 = jnp.concatenate(taps2, axis=1)
        gb = jnp.dot(col2, w2,
                     preferred_element_type=jnp.float32)
        gamma1 = gb[:, :c] + (1.0 + bg_ref[...])
        beta = gb[:, c:] + bb_ref[...]

        x = x_ref[bi]
        mean = jnp.mean(x, axis=0, keepdims=True)
        diff = x - mean
        var = jnp.mean(diff * diff, axis=0, keepdims=True)
        xhat = diff * jax.lax.rsqrt(var + _EPS)
        o_ref[bi] = xhat * gamma1 + beta


def kernel(x, segmap, w1, b1, wg, bg, wb, bb):
    B, C, H, W = x.shape
    nhidden, label_nc = w1.shape[0], w1.shape[1]
    Hs, Ws = segmap.shape[2], segmap.shape[3]
    HW = H * W
    halo = W + 1
    assert H % Hs == 0 and W % Ws == 0, "kernel assumes integer upsample"

    xt = x.transpose(0, 2, 3, 1).reshape(B, HW, C)
    w1t = w1.transpose(2, 3, 0, 1).reshape(9, nhidden, label_nc)
    wgt = wg.transpose(2, 3, 0, 1).reshape(9, C, nhidden)
    wbt = wb.transpose(2, 3, 0, 1).reshape(9, C, nhidden)

    nb = 1
    out = pl.pallas_call(
        functools.partial(_fused_spade_kernel, width=W, halo=halo),
        out_shape=jax.ShapeDtypeStruct((B, HW, C), x.dtype),
        grid=(B // nb,),
        in_specs=[
            pl.BlockSpec((nb, label_nc, Hs, Ws), lambda b: (b, 0, 0, 0)),
            pl.BlockSpec((9, nhidden, label_nc), lambda b: (0, 0, 0)),
            pl.BlockSpec((nhidden, 1), lambda b: (0, 0)),
            pl.BlockSpec((9, C, nhidden), lambda b: (0, 0, 0)),
            pl.BlockSpec((9, C, nhidden), lambda b: (0, 0, 0)),
            pl.BlockSpec((1, C), lambda b: (0, 0)),
            pl.BlockSpec((1, C), lambda b: (0, 0)),
            pl.BlockSpec((nb, HW, C), lambda b: (b, 0, 0)),
        ],
        out_specs=pl.BlockSpec((nb, HW, C), lambda b: (b, 0, 0)),
        scratch_shapes=[
            pltpu.VMEM((nhidden, 9 * label_nc), jnp.bfloat16),
            pltpu.VMEM((9 * nhidden, 2 * C), jnp.bfloat16),
            pltpu.VMEM((3, HW + 2 * halo), jnp.bfloat16),
            pltpu.VMEM((HW, 2), jnp.bfloat16),
            pltpu.VMEM((Ws, W), jnp.bfloat16),
        ],
        compiler_params=pltpu.CompilerParams(
            dimension_semantics=("arbitrary",),
            vmem_limit_bytes=60 * 1024 * 1024),
    )(segmap, w1t, b1.reshape(nhidden, 1), wgt, wbt, bg.reshape(1, C),
      bb.reshape(1, C), xt)

    return out.reshape(B, H, W, C).transpose(0, 3, 1, 2)


# speedup vs baseline: 1.0394x; 1.0394x over previous
import functools

import jax
import jax.numpy as jnp
from jax.experimental import pallas as pl
from jax.experimental.pallas import tpu as pltpu

_EPS = 1e-5


def _eye_bf16(n):
    r = jax.lax.broadcasted_iota(jnp.int32, (n, n), 0)
    c = jax.lax.broadcasted_iota(jnp.int32, (n, n), 1)
    return (r == c).astype(jnp.bfloat16)


def _mxu_t(a):
    eye = _eye_bf16(a.shape[0])
    return jax.lax.dot_general(
        a, eye, (((0,), (0,)), ((), ())),
        preferred_element_type=jnp.float32).astype(jnp.bfloat16)


def _fused_spade_kernel(seg_ref, w1_ref, b1_ref, wg_ref, wb_ref, bg_ref,
                        bb_ref, x_ref, o_ref, *, width, halo):
    nb, label_nc, hs, ws = seg_ref.shape
    _, hw, c = x_ref.shape
    nh = w1_ref.shape[1]
    npad = hw + 2 * halo

    fh, fw = (hw // width) // hs, width // ws
    liota = jax.lax.broadcasted_iota(jnp.int32, (ws, width), 1)
    qiota = jax.lax.broadcasted_iota(jnp.int32, (ws, width), 0)
    expand = (liota // fw == qiota).astype(jnp.bfloat16)

    p = jax.lax.broadcasted_iota(jnp.int32, (1, npad), 1) - halo
    pw = jax.lax.rem(p, width)
    interior = ((p >= 0) & (p < hw)).astype(jnp.bfloat16)
    nl1 = (pw != 0).astype(jnp.bfloat16)
    nr1 = jnp.logical_and(pw != width - 1, pw != -1).astype(jnp.bfloat16)
    j = jax.lax.broadcasted_iota(jnp.int32, (hw, 1), 0)
    jw = jax.lax.rem(j, width)
    nl2 = (jw != 0).astype(jnp.bfloat16)
    nr2 = (jw != width - 1).astype(jnp.bfloat16)

    w1c = jnp.concatenate(
        [w1_ref[k].astype(jnp.bfloat16) for k in range(9)], axis=1)

    wgbc = jnp.concatenate(
        [jnp.concatenate([wg_ref[k].astype(jnp.bfloat16),
                          wb_ref[k].astype(jnp.bfloat16)], axis=0)
         for k in range(9)], axis=1)
    w2 = _mxu_t(wgbc)

    for bi in range(nb):
        s = seg_ref[bi].astype(jnp.bfloat16)
        d = jnp.dot(s.reshape(label_nc * hs, ws), expand,
                    preferred_element_type=jnp.float32).astype(jnp.bfloat16)
        d = d.reshape(label_nc, hs, width)
        e = jnp.concatenate([d] * fh, axis=2)
        seg = e.reshape(label_nc, hw)
        seg = jnp.pad(seg, ((0, 0), (2 * halo, 2 * halo)))

        taps1 = []
        for kh in range(3):
            for kw in range(3):
                off = (kh - 1) * width + (kw - 1)
                tap = seg[:, halo + off: halo + off + npad]
                if kw == 0:
                    tap = tap * nl1
                elif kw == 2:
                    tap = tap * nr1
                taps1.append(tap)
        col1 = jnp.concatenate(taps1, axis=0)
        acc1 = jnp.dot(w1c, col1, preferred_element_type=jnp.float32)
        actv = jnp.maximum(acc1 + b1_ref[...], 0.0).astype(jnp.bfloat16)
        actv = actv * interior
        actv_t = _mxu_t(actv)

        taps2 = []
        for kh in range(3):
            for kw in range(3):
                off = (kh - 1) * width + (kw - 1)
                tap = actv_t[halo + off: halo + off + hw, :]
                if kw == 0:
                    tap = tap * nl2
                elif kw == 2:
                    tap = tap * nr2
                taps2.append(tap)
        col2 = jnp.concatenate(taps2, axis=1)
        gb = jnp.dot(col2, w2,
                     preferred_element_type=jnp.float32)
        gamma1 = gb[:, :c] + (1.0 + bg_ref[...])
        beta = gb[:, c:] + bb_ref[...]

        x = x_ref[bi]
        mean = jnp.mean(x, axis=0, keepdims=True)
        diff = x - mean
        var = jnp.mean(diff * diff, axis=0, keepdims=True)
        xhat = diff * jax.lax.rsqrt(var + _EPS)
        o_ref[bi] = xhat * gamma1 + beta


def kernel(x, segmap, w1, b1, wg, bg, wb, bb):
    B, C, H, W = x.shape
    nhidden, label_nc = w1.shape[0], w1.shape[1]
    Hs, Ws = segmap.shape[2], segmap.shape[3]
    HW = H * W
    halo = W + 1
    assert H % Hs == 0 and W % Ws == 0, "kernel assumes integer upsample"

    xt = x.transpose(0, 2, 3, 1).reshape(B, HW, C)
    w1t = w1.transpose(2, 3, 0, 1).reshape(9, nhidden, label_nc)
    wgt = wg.transpose(2, 3, 0, 1).reshape(9, C, nhidden)
    wbt = wb.transpose(2, 3, 0, 1).reshape(9, C, nhidden)

    nb = 1
    out = pl.pallas_call(
        functools.partial(_fused_spade_kernel, width=W, halo=halo),
        out_shape=jax.ShapeDtypeStruct((B, HW, C), x.dtype),
        grid=(B // nb,),
        in_specs=[
            pl.BlockSpec((nb, label_nc, Hs, Ws), lambda b: (b, 0, 0, 0)),
            pl.BlockSpec((9, nhidden, label_nc), lambda b: (0, 0, 0)),
            pl.BlockSpec((nhidden, 1), lambda b: (0, 0)),
            pl.BlockSpec((9, C, nhidden), lambda b: (0, 0, 0)),
            pl.BlockSpec((9, C, nhidden), lambda b: (0, 0, 0)),
            pl.BlockSpec((1, C), lambda b: (0, 0)),
            pl.BlockSpec((1, C), lambda b: (0, 0)),
            pl.BlockSpec((nb, HW, C), lambda b: (b, 0, 0)),
        ],
        out_specs=pl.BlockSpec((nb, HW, C), lambda b: (b, 0, 0)),
        compiler_params=pltpu.CompilerParams(
            dimension_semantics=("parallel",),
            vmem_limit_bytes=60 * 1024 * 1024),
    )(segmap, w1t, b1.reshape(nhidden, 1), wgt, wbt, bg.reshape(1, C),
      bb.reshape(1, C), xt)

    return out.reshape(B, H, W, C).transpose(0, 3, 1, 2)
